# baseline (speedup 1.0000x reference)
import functools

import jax
import jax.numpy as jnp
from jax import lax
from jax.experimental import pallas as pl
from jax.experimental.pallas import tpu as pltpu
from jax.experimental.pallas import tpu_sc as plsc

B = 16384
DIM = 32
LANES = 16
PACK = 128 // DIM

_info = plsc.get_sparse_core_info()
_NC, _NS = _info.num_cores, _info.num_subcores
_NW = _NC * _NS
_BPW = B // _NW
_NCHUNK = 4
_CHUNK = _BPW // _NCHUNK
_NBUF = 2


def _mf_body(u_hbm, i_hbm, U_hbm, V_hbm, bu_hbm, bi_hbm, mu_hbm, out_hbm,
             u_idx, i_idx, u_big, i_big, u_off, i_off,
             u_rows, v_rows, bu_rows, bi_rows, mu_v, out_v, sem, bsem):
    wid = lax.axis_index("s") * _NC + lax.axis_index("c")
    base = wid * _BPW

    for c in range(_NCHUNK):
        pltpu.sync_copy(u_hbm.at[pl.ds(base + c * _CHUNK, _CHUNK)], u_idx.at[c])
        pltpu.sync_copy(i_hbm.at[pl.ds(base + c * _CHUNK, _CHUNK)], i_idx.at[c])
    pltpu.sync_copy(mu_hbm, mu_v.at[pl.ds(0, 1)])
    for c in range(_NCHUNK):
        for k in range(_CHUNK // LANES):
            sl = pl.ds(k * LANES, LANES)
            uv = u_idx.at[c][sl]
            iv = i_idx.at[c][sl]
            u_big.at[c][sl] = lax.shift_right_logical(uv, 2)
            i_big.at[c][sl] = lax.shift_right_logical(iv, 2)
            u_off[pl.ds(c * _CHUNK + k * LANES, LANES)] = lax.shift_left(uv & 3, 5)
            i_off[pl.ds(c * _CHUNK + k * LANES, LANES)] = lax.shift_left(iv & 3, 5)

    bias_cps = []
    for c in range(_NCHUNK):
        bias_cps.append(pltpu.async_copy(bu_hbm.at[u_idx.at[c]], bu_rows.at[c], bsem))
        bias_cps.append(pltpu.async_copy(bi_hbm.at[i_idx.at[c]], bi_rows.at[c], bsem))

    def fire(c, buf):
        cp_u = pltpu.async_copy(U_hbm.at[u_big.at[c]], u_rows.at[buf], sem)
        cp_v = pltpu.async_copy(V_hbm.at[i_big.at[c]], v_rows.at[buf], sem)
        return cp_u, cp_v

    pend = [fire(0, 0)]

    mu_s = mu_v[...][0]
    lane = lax.iota(jnp.int32, LANES)

    for c in range(_NCHUNK):
        if c + 1 < _NCHUNK:
            pend.append(fire(c + 1, (c + 1) % _NBUF))
        for cp in pend.pop(0):
            cp.wait()
        buf = c % _NBUF
        for k in range(_CHUNK // LANES):
            r0 = k * LANES
            rows = r0 + lane
            colu = u_off[pl.ds(c * _CHUNK + r0, LANES)]
            colv = i_off[pl.ds(c * _CHUNK + r0, LANES)]
            acc = (bu_rows.at[c][pl.ds(r0, LANES)]
                   + bi_rows.at[c][pl.ds(r0, LANES)] + mu_s)
            for d in range(DIM):
                acc = acc + (plsc.load_gather(u_rows, [jnp.full((LANES,), buf, jnp.int32), rows, colu + d])
                             * plsc.load_gather(v_rows, [jnp.full((LANES,), buf, jnp.int32), rows, colv + d]))
            out_v[pl.ds(c * _CHUNK + r0, LANES)] = acc

    for cp in bias_cps:
        cp.wait()
    pltpu.sync_copy(out_v, out_hbm.at[pl.ds(base, _BPW)])


@jax.jit
def _mf_sc(u, i, U, V, bu, bi, mu):
    mesh = plsc.VectorSubcoreMesh(core_axis_name="c", subcore_axis_name="s")
    Up = U.reshape(U.shape[0] // PACK, 128)
    Vp = V.reshape(V.shape[0] // PACK, 128)
    return pl.kernel(
        _mf_body,
        mesh=mesh,
        compiler_params=pltpu.CompilerParams(needs_layout_passes=False),
        out_type=jax.ShapeDtypeStruct((B,), jnp.float32),
        scratch_types=[
            pltpu.VMEM((_NCHUNK, _CHUNK), jnp.int32),
            pltpu.VMEM((_NCHUNK, _CHUNK), jnp.int32),
            pltpu.VMEM((_NCHUNK, _CHUNK), jnp.int32),
            pltpu.VMEM((_NCHUNK, _CHUNK), jnp.int32),
            pltpu.VMEM((_BPW,), jnp.int32),
            pltpu.VMEM((_BPW,), jnp.int32),
            pltpu.VMEM((_NBUF, _CHUNK, 128), jnp.float32),
            pltpu.VMEM((_NBUF, _CHUNK, 128), jnp.float32),
            pltpu.VMEM((_NCHUNK, _CHUNK), jnp.float32),
            pltpu.VMEM((_NCHUNK, _CHUNK), jnp.float32),
            pltpu.VMEM((LANES,), jnp.float32),
            pltpu.VMEM((_BPW,), jnp.float32),
            pltpu.SemaphoreType.DMA,
            pltpu.SemaphoreType.DMA,
        ],
    )(u, i, Up, Vp, bu.reshape(-1), bi.reshape(-1), mu)


def kernel(u, i, U, V, bu, bi, mu):
    return _mf_sc(u, i, U, V, bu, bi, mu)


# speedup vs baseline: 1.0201x; 1.0201x over previous
import jax
import jax.numpy as jnp
from jax import lax
from jax.experimental import pallas as pl
from jax.experimental.pallas import tpu as pltpu
from jax.experimental.pallas import tpu_sc as plsc

B = 16384
DIM = 32
LANES = 16
NROWS = 1000000

_info = plsc.get_sparse_core_info()
_NC, _NS = _info.num_cores, _info.num_subcores
_NW = _NC * _NS
_BPW = B // _NW
_NCHUNK = 4
_CHUNK = _BPW // _NCHUNK


def _mf_body(u_hbm, i_hbm, U_hbm, V_hbm, bu_hbm, bi_hbm, mu_hbm, out_hbm,
             u_idx, i_idx, u_rows, v_rows, bu_rows, bi_rows, mu_v, out_v,
             sem, bsem):
    wid = lax.axis_index("s") * _NC + lax.axis_index("c")
    base = wid * _BPW

    for c in range(_NCHUNK):
        pltpu.sync_copy(u_hbm.at[pl.ds(base + c * _CHUNK, _CHUNK)], u_idx.at[c])
        pltpu.sync_copy(i_hbm.at[pl.ds(base + c * _CHUNK, _CHUNK)], i_idx.at[c])
    pltpu.sync_copy(mu_hbm, mu_v.at[pl.ds(0, 1)])

    cps = []
    for c in range(_NCHUNK):
        cps.append(pltpu.async_copy(
            U_hbm.at[u_idx.at[c]], u_rows.at[pl.ds(c * _CHUNK, _CHUNK)], sem))
        cps.append(pltpu.async_copy(
            V_hbm.at[i_idx.at[c]], v_rows.at[pl.ds(c * _CHUNK, _CHUNK)], sem))
        cps.append(pltpu.async_copy(bu_hbm.at[u_idx.at[c]], bu_rows.at[c], bsem))
        cps.append(pltpu.async_copy(bi_hbm.at[i_idx.at[c]], bi_rows.at[c], bsem))
    for cp in cps:
        cp.wait()

    lane = lax.iota(jnp.int32, LANES)

    def step(g, carry):
        acc = jnp.zeros((LANES,), jnp.float32)
        for j in range(LANES):
            r = g * LANES + j
            ur = u_rows.at[r]
            vr = v_rows.at[r]
            p = (ur[pl.ds(0, LANES)] * vr[pl.ds(0, LANES)]
                 + ur[pl.ds(LANES, LANES)] * vr[pl.ds(LANES, LANES)])
            acc = jnp.where(lane == j, jnp.sum(p), acc)
        out_v[pl.ds(g * LANES, LANES)] = acc
        return carry

    lax.fori_loop(0, _BPW // LANES, step, 0)

    mu_s = mu_v[...][0]
    for c in range(_NCHUNK):
        for k in range(_CHUNK // LANES):
            sl = pl.ds(c * _CHUNK + k * LANES, LANES)
            ksl = pl.ds(k * LANES, LANES)
            out_v[sl] = (out_v[sl] + bu_rows.at[c][ksl]
                         + bi_rows.at[c][ksl] + mu_s)

    pltpu.sync_copy(out_v, out_hbm.at[pl.ds(base, _BPW)])


@jax.jit
def _mf_sc(u, i, U, V, bu, bi, mu):
    mesh = plsc.VectorSubcoreMesh(core_axis_name="c", subcore_axis_name="s")
    return pl.kernel(
        _mf_body,
        mesh=mesh,
        compiler_params=pltpu.CompilerParams(needs_layout_passes=False,
                                             use_tc_tiling_on_sc=False),
        out_type=jax.ShapeDtypeStruct((B,), jnp.float32),
        scratch_types=[
            pltpu.VMEM((_NCHUNK, _CHUNK), jnp.int32),
            pltpu.VMEM((_NCHUNK, _CHUNK), jnp.int32),
            pltpu.VMEM((_BPW, DIM), jnp.float32),
            pltpu.VMEM((_BPW, DIM), jnp.float32),
            pltpu.VMEM((_NCHUNK, _CHUNK), jnp.float32),
            pltpu.VMEM((_NCHUNK, _CHUNK), jnp.float32),
            pltpu.VMEM((LANES,), jnp.float32),
            pltpu.VMEM((_BPW,), jnp.float32),
            pltpu.SemaphoreType.DMA,
            pltpu.SemaphoreType.DMA,
        ],
    )(u, i, U, V, bu.reshape(-1), bi.reshape(-1), mu)


def kernel(u, i, U, V, bu, bi, mu):
    return _mf_sc(u, i, U, V, bu, bi, mu)
